# baseline (speedup 1.0000x reference)
import jax
import jax.numpy as jnp
from jax import lax
from jax.experimental import pallas as pl
from jax.experimental.pallas import tpu as pltpu
from jax.experimental.pallas import tpu_sc as plsc

N = 10000
D = 128
H = 128
G = 64
E = 320000
BN_EPS = 1e-5

NC = 2
NS = 16
NW = NC * NS
CHUNK = 80
NSLOT = 4
NCHUNK0 = 196
NCHUNK1 = 60
NCHUNK_DEG = 128
EPT_DEG = NCHUNK_DEG * CHUNK
SC0_EDGES = NS * NCHUNK0 * CHUNK
EPAD = SC0_EDGES + NS * NCHUNK1 * CHUNK
NPAD = N + 112
RPT = NPAD // NS
DEGW = 16


def _fill_idx(dstbuf, bulk, i):
    base = i * CHUNK
    for j in range(CHUNK // 16):
        dstbuf[pl.ds(j * 16, 16)] = bulk[pl.ds(base + j * 16, 16)]


def _sc_degree_body(dst_hbm, ones_hbm, zeros_hbm, out_hbm,
                    didx_all, didx_s, ones_v, ssems, acc_sh):
    cid = lax.axis_index("c")
    sid = lax.axis_index("s")
    wid = cid * NS + sid
    pltpu.sync_copy(zeros_hbm.at[pl.ds(sid * RPT, RPT)],
                    acc_sh.at[pl.ds(sid * RPT, RPT)])
    pltpu.sync_copy(ones_hbm, ones_v)
    pltpu.sync_copy(dst_hbm.at[pl.ds(wid * EPT_DEG, EPT_DEG)], didx_all)
    plsc.subcore_barrier()

    def scatter(k):
        pltpu.async_copy(ones_v, acc_sh.at[didx_s[k]], ssems[k], add=True)

    def wait_scatter(k):
        pltpu.make_async_copy(ones_v, acc_sh.at[didx_s[k]],
                              ssems[k]).wait()

    for k in range(NSLOT):
        _fill_idx(didx_s[k], didx_all, k)
        scatter(k)

    def group(g, carry):
        for k in range(NSLOT):
            i = g * NSLOT + k

            @pl.when(g < NCHUNK_DEG // NSLOT - 1)
            def _():
                wait_scatter(k)
                _fill_idx(didx_s[k], didx_all, i + NSLOT)
                scatter(k)
        return carry

    lax.fori_loop(0, NCHUNK_DEG // NSLOT, group, 0)
    for k in range(NSLOT):
        wait_scatter(k)
    plsc.subcore_barrier()
    pltpu.sync_copy(acc_sh.at[pl.ds(sid * RPT, RPT)],
                    out_hbm.at[cid, pl.ds(sid * RPT, RPT)])


def _sc_agg_body(src_hbm, dst_hbm, m_hbm, zeros_hbm, out_hbm,
                 sidx_s, didx_s, bufs, gsems, ssems, isgs, isds,
                 acc_sh):
    cid = lax.axis_index("c")
    sid = lax.axis_index("s")
    pltpu.sync_copy(zeros_hbm.at[pl.ds(sid * RPT, RPT)],
                    acc_sh.at[pl.ds(sid * RPT, RPT)])

    def gather(k):
        pltpu.async_copy(m_hbm.at[sidx_s[k]], bufs[k], gsems[k])

    def wait_gather(k):
        pltpu.make_async_copy(m_hbm.at[sidx_s[k]], bufs[k],
                              gsems[k]).wait()

    def scatter(k):
        pltpu.async_copy(bufs[k], acc_sh.at[didx_s[k]], ssems[k],
                         add=True)

    def wait_scatter(k):
        pltpu.make_async_copy(bufs[k], acc_sh.at[didx_s[k]],
                              ssems[k]).wait()

    def ring(nchunk, ebase):
        def start_sidx(i, k):
            pltpu.async_copy(src_hbm.at[pl.ds(ebase + i * CHUNK, CHUNK)],
                             sidx_s[k], isgs[k])

        def wait_sidx(k):
            pltpu.make_async_copy(src_hbm.at[pl.ds(ebase, CHUNK)],
                                  sidx_s[k], isgs[k]).wait()

        def start_didx(i, k):
            pltpu.async_copy(dst_hbm.at[pl.ds(ebase + i * CHUNK, CHUNK)],
                             didx_s[k], isds[k])

        def wait_didx(k):
            pltpu.make_async_copy(dst_hbm.at[pl.ds(ebase, CHUNK)],
                                  didx_s[k], isds[k]).wait()

        for k in range(NSLOT):
            start_didx(k, k)
            start_sidx(k, k)
            wait_sidx(k)
            gather(k)

        def group(g, carry):
            for k in range(NSLOT):
                i = g * NSLOT + k
                wait_gather(k)
                wait_didx(k)
                scatter(k)

                @pl.when(g < nchunk // NSLOT - 1)
                def _():
                    start_sidx(i + NSLOT, k)
                    wait_scatter(k)
                    start_didx(i + NSLOT, k)
                    wait_sidx(k)
                    gather(k)
            return carry

        lax.fori_loop(0, nchunk // NSLOT, group, 0)
        for k in range(NSLOT):
            wait_scatter(k)

    plsc.subcore_barrier()

    @pl.when(cid == 0)
    def _():
        ring(NCHUNK0, sid * (NCHUNK0 * CHUNK))

    @pl.when(cid == 1)
    def _():
        ring(NCHUNK1, SC0_EDGES + sid * (NCHUNK1 * CHUNK))

    plsc.subcore_barrier()
    pltpu.sync_copy(acc_sh.at[pl.ds(sid * RPT, RPT)],
                    out_hbm.at[cid, pl.ds(sid * RPT, RPT)])


def _sc_mesh():
    return plsc.VectorSubcoreMesh(core_axis_name="c", subcore_axis_name="s",
                                  num_cores=NC, num_subcores=NS)


def _sc_degree(dstp, ones_rows, zeros_m):
    return pl.kernel(
        _sc_degree_body,
        out_type=jax.ShapeDtypeStruct((NC, NPAD, H), jnp.float32),
        mesh=_sc_mesh(),
        scratch_types=[
            pltpu.VMEM((EPT_DEG,), jnp.int32),
            tuple(pltpu.VMEM((CHUNK,), jnp.int32) for _ in range(NSLOT)),
            pltpu.VMEM((CHUNK, H), jnp.float32),
            tuple(pltpu.SemaphoreType.DMA for _ in range(NSLOT)),
            pltpu.VMEM_SHARED((NPAD, H), jnp.float32),
        ],
    )(dstp, ones_rows, zeros_m)


def _sc_agg(srcp, dstp, mpad, zeros_m):
    return pl.kernel(
        _sc_agg_body,
        out_type=jax.ShapeDtypeStruct((NC, NPAD, H), jnp.float32),
        mesh=_sc_mesh(),
        scratch_types=[
            tuple(pltpu.VMEM((CHUNK,), jnp.int32) for _ in range(NSLOT)),
            tuple(pltpu.VMEM((CHUNK,), jnp.int32) for _ in range(NSLOT)),
            tuple(pltpu.VMEM((CHUNK, H), jnp.float32) for _ in range(NSLOT)),
            tuple(pltpu.SemaphoreType.DMA for _ in range(NSLOT)),
            tuple(pltpu.SemaphoreType.DMA for _ in range(NSLOT)),
            tuple(pltpu.SemaphoreType.DMA for _ in range(NSLOT)),
            tuple(pltpu.SemaphoreType.DMA for _ in range(NSLOT)),
            pltpu.VMEM_SHARED((NPAD, H), jnp.float32),
        ],
    )(srcp, dstp, mpad, zeros_m)


def _tc_dinv_body(pdeg_ref, dinv_ref):
    deg = pdeg_ref[0, :, 0:1] + pdeg_ref[1, :, 0:1] + 1.0
    dinv_ref[...] = lax.rsqrt(jnp.maximum(deg, 1e-12))


def _tc_pre_body(x_ref, w_ref, dinv_ref, m_ref):
    dinv = dinv_ref[0:N, :]
    h = jnp.dot(x_ref[...], w_ref[...], preferred_element_type=jnp.float32,
                 precision=lax.Precision.HIGHEST)
    m_ref[0:N, :] = h * dinv
    m_ref[N:NPAD, :] = jnp.zeros((NPAD - N, H), jnp.float32)


def _conv_bn(m_ref, ps_ref, dinv, b_ref, g_ref, be_ref):
    t = (m_ref[0:N, :] + ps_ref[0, 0:N, :] + ps_ref[1, 0:N, :]) * dinv + b_ref[...]
    t = jnp.maximum(t, 0.0)
    mean = jnp.mean(t, axis=0, keepdims=True)
    var = jnp.mean((t - mean) ** 2, axis=0, keepdims=True)
    return g_ref[...] * (t - mean) * lax.rsqrt(var + BN_EPS) + be_ref[...]


def _tc_mid_body(m_ref, ps_ref, dinv_ref, b_ref, g_ref, be_ref, w2_ref,
                 z_ref, m2_ref):
    dinv = dinv_ref[0:N, :]
    z = _conv_bn(m_ref, ps_ref, dinv, b_ref, g_ref, be_ref)
    z_ref[...] = z
    h2 = jnp.dot(z, w2_ref[...], preferred_element_type=jnp.float32,
                 precision=lax.Precision.HIGHEST)
    m2_ref[0:N, :] = h2 * dinv
    m2_ref[N:NPAD, :] = jnp.zeros((NPAD - N, H), jnp.float32)


def _tc_final_body(m_ref, ps_ref, dinv_ref, b_ref, g_ref, be_ref,
                   z1_ref, batch_ref, zn_ref, gn_ref):
    dinv = dinv_ref[0:N, :]
    z2 = _conv_bn(m_ref, ps_ref, dinv, b_ref, g_ref, be_ref)
    z1 = z1_ref[...]

    nrm = jnp.sqrt(jnp.sum(z1 * z1, axis=1, keepdims=True)
                   + jnp.sum(z2 * z2, axis=1, keepdims=True))
    rinv = 1.0 / jnp.maximum(nrm, 1e-12)
    zn_ref[:, 0:H] = z1 * rinv
    zn_ref[:, H:2 * H] = z2 * rinv

    pt = (batch_ref[...] == lax.broadcasted_iota(jnp.int32, (G, N), 0)
          ).astype(jnp.float32)
    cnt = jnp.dot(pt, jnp.ones((N, 1), jnp.float32),
                  preferred_element_type=jnp.float32,
                 precision=lax.Precision.HIGHEST)
    s1 = jnp.dot(pt, z1, preferred_element_type=jnp.float32,
                 precision=lax.Precision.HIGHEST)
    s2 = jnp.dot(pt, z2, preferred_element_type=jnp.float32,
                 precision=lax.Precision.HIGHEST)
    cinv = 1.0 / jnp.maximum(cnt, 1.0)
    gm1 = s1 * cinv
    gm2 = s2 * cinv
    gnrm = jnp.sqrt(jnp.sum(gm1 * gm1, axis=1, keepdims=True)
                    + jnp.sum(gm2 * gm2, axis=1, keepdims=True))
    grinv = 1.0 / jnp.maximum(gnrm, 1e-12)
    gn_ref[:, 0:H] = gm1 * grinv
    gn_ref[:, H:2 * H] = gm2 * grinv


def _f32(shape):
    return jax.ShapeDtypeStruct(shape, jnp.float32)


def kernel(x, edge_index, batch, W1, b1, gamma1, beta1, W2, b2, gamma2, beta2):
    src = edge_index[0]
    dst = edge_index[1]
    pad = EPAD - E
    srcp = jnp.concatenate([src, jnp.full((pad,), N, jnp.int32)])
    dstp = jnp.concatenate([dst, jnp.full((pad,), N, jnp.int32)])
    zeros_m = jnp.zeros((NPAD, H), jnp.float32)
    ones_rows = jnp.ones((CHUNK, H), jnp.float32)

    b1r = b1.reshape(1, H)
    g1r = gamma1.reshape(1, H)
    be1r = beta1.reshape(1, H)
    b2r = b2.reshape(1, H)
    g2r = gamma2.reshape(1, H)
    be2r = beta2.reshape(1, H)
    batch_row = batch.reshape(1, N)

    pdeg = _sc_degree(dstp, ones_rows, zeros_m)

    dinv_col = pl.pallas_call(
        _tc_dinv_body, out_shape=_f32((NPAD, 1)),
    )(pdeg)

    m1 = pl.pallas_call(
        _tc_pre_body, out_shape=_f32((NPAD, H)),
    )(x.astype(jnp.float32), W1, dinv_col)

    ps1 = _sc_agg(srcp, dstp, m1, zeros_m)

    z1, m2 = pl.pallas_call(
        _tc_mid_body, out_shape=(_f32((N, H)), _f32((NPAD, H))),
    )(m1, ps1, dinv_col, b1r, g1r, be1r, W2)

    ps2 = _sc_agg(srcp, dstp, m2, zeros_m)

    zn, gn = pl.pallas_call(
        _tc_final_body, out_shape=(_f32((N, 2 * H)), _f32((G, 2 * H))),
    )(m2, ps2, dinv_col, b2r, g2r, be2r, z1, batch_row)

    return zn, gn


# speedup vs baseline: 1.0745x; 1.0745x over previous
import jax
import jax.numpy as jnp
from jax import lax
from jax.experimental import pallas as pl
from jax.experimental.pallas import tpu as pltpu
from jax.experimental.pallas import tpu_sc as plsc

N = 10000
D = 128
H = 128
G = 64
E = 320000
BN_EPS = 1e-5

NC = 2
NS = 16
NW = NC * NS
CHUNK = 128
NCHUNK = 79
EPT = NCHUNK * CHUNK
EPAD = NW * EPT
NPAD = N + 112
RPT = NPAD // NS
DEGW = 16


def _sc_degree_body(dst_hbm, ones_hbm, zeros_hbm, out_hbm, didx_v, ones_v, acc_sh):
    cid = lax.axis_index("c")
    sid = lax.axis_index("s")
    wid = cid * NS + sid
    pltpu.sync_copy(zeros_hbm.at[pl.ds(sid * RPT, RPT)],
                    acc_sh.at[pl.ds(sid * RPT, RPT)])
    pltpu.sync_copy(ones_hbm, ones_v)
    plsc.subcore_barrier()

    ebase = wid * EPT

    def chunk(i, carry):
        base = ebase + i * CHUNK
        pltpu.sync_copy(dst_hbm.at[pl.ds(base, CHUNK)], didx_v)
        pltpu.sync_copy(ones_v, acc_sh.at[didx_v], add=True)
        return carry

    lax.fori_loop(0, NCHUNK, chunk, 0)
    plsc.subcore_barrier()
    pltpu.sync_copy(acc_sh.at[pl.ds(sid * RPT, RPT)],
                    out_hbm.at[cid, pl.ds(sid * RPT, RPT)])


def _sc_agg_body(src_hbm, dst_hbm, m_hbm, zeros_hbm, out_hbm,
                 sidx_v, didx_v, rows_v, acc_sh):
    cid = lax.axis_index("c")
    sid = lax.axis_index("s")
    wid = cid * NS + sid
    pltpu.sync_copy(zeros_hbm.at[pl.ds(sid * RPT, RPT)],
                    acc_sh.at[pl.ds(sid * RPT, RPT)])
    plsc.subcore_barrier()

    ebase = wid * EPT

    def chunk(i, carry):
        base = ebase + i * CHUNK
        pltpu.sync_copy(src_hbm.at[pl.ds(base, CHUNK)], sidx_v)
        pltpu.sync_copy(dst_hbm.at[pl.ds(base, CHUNK)], didx_v)
        pltpu.sync_copy(m_hbm.at[sidx_v], rows_v)
        pltpu.sync_copy(rows_v, acc_sh.at[didx_v], add=True)
        return carry

    lax.fori_loop(0, NCHUNK, chunk, 0)
    plsc.subcore_barrier()
    pltpu.sync_copy(acc_sh.at[pl.ds(sid * RPT, RPT)],
                    out_hbm.at[cid, pl.ds(sid * RPT, RPT)])


def _sc_mesh():
    return plsc.VectorSubcoreMesh(core_axis_name="c", subcore_axis_name="s",
                                  num_cores=NC, num_subcores=NS)


def _sc_degree(dstp, ones_rows, zeros_m):
    return pl.kernel(
        _sc_degree_body,
        out_type=jax.ShapeDtypeStruct((NC, NPAD, H), jnp.float32),
        mesh=_sc_mesh(),
        scratch_types=[
            pltpu.VMEM((CHUNK,), jnp.int32),
            pltpu.VMEM((CHUNK, H), jnp.float32),
            pltpu.VMEM_SHARED((NPAD, H), jnp.float32),
        ],
    )(dstp, ones_rows, zeros_m)


def _sc_agg(srcp, dstp, mpad, zeros_m):
    return pl.kernel(
        _sc_agg_body,
        out_type=jax.ShapeDtypeStruct((NC, NPAD, H), jnp.float32),
        mesh=_sc_mesh(),
        scratch_types=[
            pltpu.VMEM((CHUNK,), jnp.int32),
            pltpu.VMEM((CHUNK,), jnp.int32),
            pltpu.VMEM((CHUNK, H), jnp.float32),
            pltpu.VMEM_SHARED((NPAD, H), jnp.float32),
        ],
    )(srcp, dstp, mpad, zeros_m)


def _tc_dinv_body(pdeg_ref, dinv_ref):
    deg = pdeg_ref[0, :, 0:1] + pdeg_ref[1, :, 0:1] + 1.0
    dinv_ref[...] = lax.rsqrt(jnp.maximum(deg, 1e-12))


def _tc_pre_body(x_ref, w_ref, dinv_ref, m_ref):
    dinv = dinv_ref[0:N, :]
    h = jnp.dot(x_ref[...], w_ref[...], preferred_element_type=jnp.float32,
                 precision=lax.Precision.HIGHEST)
    m_ref[0:N, :] = h * dinv
    m_ref[N:NPAD, :] = jnp.zeros((NPAD - N, H), jnp.float32)


def _conv_bn(m_ref, ps_ref, dinv, b_ref, g_ref, be_ref):
    t = (m_ref[0:N, :] + ps_ref[0, 0:N, :] + ps_ref[1, 0:N, :]) * dinv + b_ref[...]
    t = jnp.maximum(t, 0.0)
    mean = jnp.mean(t, axis=0, keepdims=True)
    var = jnp.mean((t - mean) ** 2, axis=0, keepdims=True)
    return g_ref[...] * (t - mean) * lax.rsqrt(var + BN_EPS) + be_ref[...]


def _tc_mid_body(m_ref, ps_ref, dinv_ref, b_ref, g_ref, be_ref, w2_ref,
                 z_ref, m2_ref):
    dinv = dinv_ref[0:N, :]
    z = _conv_bn(m_ref, ps_ref, dinv, b_ref, g_ref, be_ref)
    z_ref[...] = z
    h2 = jnp.dot(z, w2_ref[...], preferred_element_type=jnp.float32,
                 precision=lax.Precision.HIGHEST)
    m2_ref[0:N, :] = h2 * dinv
    m2_ref[N:NPAD, :] = jnp.zeros((NPAD - N, H), jnp.float32)


def _tc_final_body(m_ref, ps_ref, dinv_ref, b_ref, g_ref, be_ref,
                   z1_ref, batch_ref, zn_ref, gn_ref):
    dinv = dinv_ref[0:N, :]
    z2 = _conv_bn(m_ref, ps_ref, dinv, b_ref, g_ref, be_ref)
    z1 = z1_ref[...]

    nrm = jnp.sqrt(jnp.sum(z1 * z1, axis=1, keepdims=True)
                   + jnp.sum(z2 * z2, axis=1, keepdims=True))
    rinv = 1.0 / jnp.maximum(nrm, 1e-12)
    zn_ref[:, 0:H] = z1 * rinv
    zn_ref[:, H:2 * H] = z2 * rinv

    pt = (batch_ref[...] == lax.broadcasted_iota(jnp.int32, (G, N), 0)
          ).astype(jnp.float32)
    cnt = jnp.dot(pt, jnp.ones((N, 1), jnp.float32),
                  preferred_element_type=jnp.float32,
                 precision=lax.Precision.HIGHEST)
    s1 = jnp.dot(pt, z1, preferred_element_type=jnp.float32,
                 precision=lax.Precision.HIGHEST)
    s2 = jnp.dot(pt, z2, preferred_element_type=jnp.float32,
                 precision=lax.Precision.HIGHEST)
    cinv = 1.0 / jnp.maximum(cnt, 1.0)
    gm1 = s1 * cinv
    gm2 = s2 * cinv
    gnrm = jnp.sqrt(jnp.sum(gm1 * gm1, axis=1, keepdims=True)
                    + jnp.sum(gm2 * gm2, axis=1, keepdims=True))
    grinv = 1.0 / jnp.maximum(gnrm, 1e-12)
    gn_ref[:, 0:H] = gm1 * grinv
    gn_ref[:, H:2 * H] = gm2 * grinv


def _f32(shape):
    return jax.ShapeDtypeStruct(shape, jnp.float32)


def kernel(x, edge_index, batch, W1, b1, gamma1, beta1, W2, b2, gamma2, beta2):
    src = edge_index[0]
    dst = edge_index[1]
    pad = EPAD - E
    srcp = jnp.concatenate([src, jnp.full((pad,), N, jnp.int32)])
    dstp = jnp.concatenate([dst, jnp.full((pad,), N, jnp.int32)])
    zeros_m = jnp.zeros((NPAD, H), jnp.float32)
    ones_rows = jnp.ones((CHUNK, H), jnp.float32)

    b1r = b1.reshape(1, H)
    g1r = gamma1.reshape(1, H)
    be1r = beta1.reshape(1, H)
    b2r = b2.reshape(1, H)
    g2r = gamma2.reshape(1, H)
    be2r = beta2.reshape(1, H)
    batch_row = batch.reshape(1, N)

    pdeg = _sc_degree(dstp, ones_rows, zeros_m)

    dinv_col = pl.pallas_call(
        _tc_dinv_body, out_shape=_f32((NPAD, 1)),
    )(pdeg)

    m1 = pl.pallas_call(
        _tc_pre_body, out_shape=_f32((NPAD, H)),
    )(x.astype(jnp.float32), W1, dinv_col)

    ps1 = _sc_agg(srcp, dstp, m1, zeros_m)

    z1, m2 = pl.pallas_call(
        _tc_mid_body, out_shape=(_f32((N, H)), _f32((NPAD, H))),
    )(m1, ps1, dinv_col, b1r, g1r, be1r, W2)

    ps2 = _sc_agg(srcp, dstp, m2, zeros_m)

    zn, gn = pl.pallas_call(
        _tc_final_body, out_shape=(_f32((N, 2 * H)), _f32((G, 2 * H))),
    )(m2, ps2, dinv_col, b2r, g2r, be2r, z1, batch_row)

    return zn, gn
